# baseline (speedup 1.0000x reference)
import functools
import math

import jax
import jax.numpy as jnp
from jax import lax
from jax.experimental import pallas as pl
from jax.experimental.pallas import tpu as pltpu
from jax.experimental.pallas import tpu_sc as plsc

VOCAB = 1000000
EMB = 64
TOKENS_SHAPE = (4096, 200)
SCALE = math.sqrt(EMB)

NC = 2
NS = 16
NW = NC * NS
LANES = 16

B = TOKENS_SHAPE[0] * TOKENS_SHAPE[1]
B_PER_W = B // NW
SUB = 128
CHUNK = 640
N_SUB = CHUNK // SUB
N_CHUNKS = B_PER_W // CHUNK


def _emb_body(tok_hbm, table_hbm, out_hbm, idx_v, rows_v, gsem):
    wid = lax.axis_index("s") * NC + lax.axis_index("c")
    tok0 = wid * B_PER_W
    out_row0 = wid * B_PER_W

    def chunk_body(g, carry):
        pltpu.sync_copy(tok_hbm.at[pl.ds(tok0 + g * CHUNK, CHUNK)], idx_v)
        copies = []
        for j in range(N_SUB):
            copies.append(
                pltpu.async_copy(
                    table_hbm.at[idx_v.at[pl.ds(j * SUB, SUB)]],
                    rows_v.at[pl.ds(j * SUB, SUB)],
                    gsem,
                )
            )
        for c in copies:
            c.wait()

        def scale_row(i, c):
            for j in range(EMB // LANES):
                sl = pl.ds(j * LANES, LANES)
                rows_v[i, sl] = rows_v[i, sl] * SCALE
            return c

        lax.fori_loop(0, CHUNK, scale_row, 0)

        pltpu.sync_copy(rows_v, out_hbm.at[pl.ds(out_row0 + g * CHUNK, CHUNK)])
        return carry

    lax.fori_loop(0, N_CHUNKS, chunk_body, 0)


_emb_call = pl.kernel(
    _emb_body,
    out_type=jax.ShapeDtypeStruct((B, EMB), jnp.float32),
    mesh=plsc.VectorSubcoreMesh(
        core_axis_name="c", subcore_axis_name="s", num_cores=NC, num_subcores=NS
    ),
    scratch_types=[
        pltpu.VMEM((CHUNK,), jnp.int32),
        pltpu.VMEM((CHUNK, EMB), jnp.float32),
        pltpu.SemaphoreType.DMA,
    ],
    compiler_params=pltpu.CompilerParams(use_tc_tiling_on_sc=False),
)


@jax.jit
def kernel(tokens, table):
    tok = tokens.reshape(B).astype(jnp.int32)
    out = _emb_call(tok, table)
    return out.reshape(TOKENS_SHAPE[0], TOKENS_SHAPE[1], EMB)
